# baseline (speedup 1.0000x reference)
import functools
import jax
import jax.numpy as jnp
from jax.experimental import pallas as pl
from jax.experimental.pallas import tpu as pltpu

E = 64
K = 2
D = 768
H = 256
O = 768
T = 512
A = T * K
CHUNK = 128


def _meta_body(idx_row_ref, w_col_ref, offs_ref, st_ref, strow_ref, sw_ref,
               pos_ref):
    f32 = jnp.float32
    i32 = jnp.int32
    idxr = idx_row_ref[...]
    iota_e = jax.lax.broadcasted_iota(i32, (E, A), 0)
    ohR = (jnp.broadcast_to(idxr, (E, A)) == iota_e).astype(f32)

    r = jax.lax.broadcasted_iota(i32, (A, A), 0)
    c = jax.lax.broadcasted_iota(i32, (A, A), 1)
    LT = (r <= c).astype(f32)
    ranksR = jax.lax.dot_general(ohR, LT, (((1,), (0,)), ((), ())),
                                 preferred_element_type=f32, precision=jax.lax.Precision.HIGHEST)
    rank_a = jnp.sum(ranksR * ohR, axis=0, keepdims=True)

    cnt_col = ranksR[:, A - 1:A]
    r64 = jax.lax.broadcasted_iota(i32, (E, E), 0)
    c64 = jax.lax.broadcasted_iota(i32, (E, E), 1)
    U = (r64 < c64).astype(f32)
    eye = (r64 == c64).astype(f32)
    offs_row = jax.lax.dot_general(cnt_col, U, (((0,), (0,)), ((), ())),
                                   preferred_element_type=f32, precision=jax.lax.Precision.HIGHEST)
    cnt_row = jax.lax.dot_general(cnt_col, eye, (((0,), (0,)), ((), ())),
                                  preferred_element_type=f32, precision=jax.lax.Precision.HIGHEST)
    offs_a = jax.lax.dot_general(offs_row, ohR, (((1,), (0,)), ((), ())),
                                 preferred_element_type=f32, precision=jax.lax.Precision.HIGHEST)
    posT = offs_a + rank_a - 1.0
    posT_i = posT.astype(i32)

    Pj = jax.lax.broadcasted_iota(i32, (A, A), 0)
    Pmat = (Pj == jnp.broadcast_to(posT_i, (A, A))).astype(f32)
    tok0 = (jax.lax.broadcasted_iota(i32, (A, 1), 0) // K).astype(f32)
    st_col = jax.lax.dot_general(Pmat, tok0, (((1,), (0,)), ((), ())),
                                 preferred_element_type=f32, precision=jax.lax.Precision.HIGHEST)
    st_row = jax.lax.dot_general(tok0, Pmat, (((0,), (1,)), ((), ())),
                                 preferred_element_type=f32, precision=jax.lax.Precision.HIGHEST)
    sw_col = jax.lax.dot_general(Pmat, w_col_ref[...], (((1,), (0,)), ((), ())),
                                 preferred_element_type=f32, precision=jax.lax.Precision.HIGHEST)

    offs_ref[0:1, :] = offs_row.astype(jnp.int32)
    offs_ref[1:2, :] = cnt_row.astype(jnp.int32)
    st_ref[...] = st_col.astype(jnp.int32)
    strow_ref[...] = st_row.astype(jnp.int32)
    sw_ref[...] = sw_col
    pos_ref[...] = posT_i


def _gather_body(st_sm, hid_ref, xs_ref):
    def row(j, carry):
        xs_ref[j, :] = hid_ref[st_sm[j], :]
        return carry
    jax.lax.fori_loop(0, A, row, 0)


def _moe_body(offs_sm, xs_ref, gu_ref, dw_ref, sw_ref, ys_ref):
    e = pl.program_id(0)

    start = offs_sm[0, e]
    cnt = offs_sm[1, e]
    end = start + cnt
    a0 = (start // CHUNK) * CHUNK
    nchunks = jnp.where(cnt > 0, (end - a0 + CHUNK - 1) // CHUNK, 0)

    @pl.when(e == 0)
    def _():
        ys_ref[...] = jnp.zeros_like(ys_ref)

    def chunk(cc, carry):
        cs = pl.multiple_of(a0 + cc * CHUNK, CHUNK)
        x = xs_ref[pl.ds(cs, CHUNK), :]
        gu = jax.lax.dot_general(
            x, gu_ref[0], (((1,), (1,)), ((), ())),
            preferred_element_type=jnp.float32)
        g = gu[:, :H]
        u = gu[:, H:]
        h = g * jax.nn.sigmoid(g) * u
        y = jax.lax.dot_general(
            h, dw_ref[0], (((1,), (1,)), ((), ())),
            preferred_element_type=jnp.float32)
        rows = cs + jax.lax.broadcasted_iota(jnp.int32, (CHUNK, 1), 0)
        scale = jnp.where((rows >= start) & (rows < end),
                          sw_ref[pl.ds(cs, CHUNK), :], 0.0)
        ys_ref[pl.ds(cs, CHUNK), :] += y * scale
        return carry

    jax.lax.fori_loop(0, nchunks, chunk, 0)


def _combine_body(pos_sm, ys_ref, out_ref):
    def row(t, carry):
        ia = pos_sm[K * t]
        ib = pos_sm[K * t + 1]
        out_ref[t, :] = ys_ref[ia, :] + ys_ref[ib, :]
        return carry
    jax.lax.fori_loop(0, T, row, 0)


@jax.jit
def kernel(hidden_states, top_k_indices, top_k_weights, gate_up_proj, down_proj):
    idx_row = top_k_indices.reshape(1, A).astype(jnp.int32)
    w_col = top_k_weights.reshape(A, 1)

    offs_cnt, st_col, st_row, sw_col, pos_row = pl.pallas_call(
        _meta_body,
        grid=(1,),
        in_specs=[
            pl.BlockSpec((1, A), lambda i: (0, 0)),
            pl.BlockSpec((A, 1), lambda i: (0, 0)),
        ],
        out_specs=[
            pl.BlockSpec((2, E), lambda i: (0, 0)),
            pl.BlockSpec((A, 1), lambda i: (0, 0)),
            pl.BlockSpec((1, A), lambda i: (0, 0)),
            pl.BlockSpec((A, 1), lambda i: (0, 0)),
            pl.BlockSpec((1, A), lambda i: (0, 0)),
        ],
        out_shape=[
            jax.ShapeDtypeStruct((2, E), jnp.int32),
            jax.ShapeDtypeStruct((A, 1), jnp.int32),
            jax.ShapeDtypeStruct((1, A), jnp.int32),
            jax.ShapeDtypeStruct((A, 1), jnp.float32),
            jax.ShapeDtypeStruct((1, A), jnp.int32),
        ],
    )(idx_row, w_col)

    st_flat = st_col.reshape(A)
    pos_flat = pos_row.reshape(A)

    x_sorted = pl.pallas_call(
        _gather_body,
        grid_spec=pltpu.PrefetchScalarGridSpec(
            num_scalar_prefetch=1,
            grid=(1,),
            in_specs=[pl.BlockSpec((T, D), lambda i, s: (0, 0))],
            out_specs=pl.BlockSpec((A, D), lambda i, s: (0, 0)),
        ),
        out_shape=jax.ShapeDtypeStruct((A, D), jnp.float32),
    )(st_flat, hidden_states)

    y_sorted = pl.pallas_call(
        _moe_body,
        grid_spec=pltpu.PrefetchScalarGridSpec(
            num_scalar_prefetch=1,
            grid=(E,),
            in_specs=[
                pl.BlockSpec((A, D), lambda e, s: (0, 0)),
                pl.BlockSpec((1, 2 * H, D), lambda e, s: (e, 0, 0)),
                pl.BlockSpec((1, O, H), lambda e, s: (e, 0, 0)),
                pl.BlockSpec((A, 1), lambda e, s: (0, 0)),
            ],
            out_specs=pl.BlockSpec((A, O), lambda e, s: (0, 0)),
        ),
        out_shape=jax.ShapeDtypeStruct((A, O), jnp.float32),
    )(offs_cnt, x_sorted, gate_up_proj, down_proj, sw_col)

    output = pl.pallas_call(
        _combine_body,
        grid_spec=pltpu.PrefetchScalarGridSpec(
            num_scalar_prefetch=1,
            grid=(1,),
            in_specs=[pl.BlockSpec((A, O), lambda i, s: (0, 0))],
            out_specs=pl.BlockSpec((T, O), lambda i, s: (0, 0)),
        ),
        out_shape=jax.ShapeDtypeStruct((T, O), jnp.float32),
    )(pos_flat, y_sorted)

    return output
